# baseline (speedup 1.0000x reference)
import jax
import jax.numpy as jnp
from jax.experimental import pallas as pl
from jax.experimental.pallas import tpu as pltpu

_N_CLUSTERS = 512
_MAX_ITER = 20
_TOL = 1e-4
_K_NEIGH = 15
_N_POINTS = 16384
_CHUNK = 8192
_N_CHUNKS = _N_POINTS // _CHUNK
_F_DIM = 128
_BEST = 128
_CAND = _BEST + _CHUNK
_NEG = -1e30
_DEAD = -2e30


def _body(pts_ref, feat_ref, cent0_ref, ct_ref, out_ref,
          closest_ref, bv_ref, bi_ref, err_ref):
    ct_ref[0] = cent0_ref[0]
    err_ref[0] = jnp.float32(1e30)

    sub512 = jax.lax.broadcasted_iota(jnp.int32, (_N_CLUSTERS, _CHUNK), 0)

    def scores_chunk(cent, k):
        off = k * _CHUNK
        xt = pts_ref[0, pl.ds(0, 3), pl.ds(off, _CHUNK)]
        dot = jax.lax.dot_general(cent, xt, (((1,), (0,)), ((), ())),
                                  preferred_element_type=jnp.float32)
        xn = (xt[0:1, :] * xt[0:1, :] + xt[2:3, :] * xt[2:3, :]) \
            + xt[1:2, :] * xt[1:2, :]
        cn = (cent[:, 0:1] * cent[:, 0:1] + cent[:, 2:3] * cent[:, 2:3]) \
            + cent[:, 1:2] * cent[:, 1:2]
        return (2.0 * dot - xn) - cn

    def kmeans_iter(it, _):
        @pl.when(err_ref[0] > _TOL)
        def _run():
            cent = ct_ref[0]

            def chunk_body(k, sums4):
                s = scores_chunk(cent, k)
                m = jnp.max(s, axis=0, keepdims=True)
                closest = jnp.min(
                    jnp.where(s >= m, sub512, _N_CLUSTERS),
                    axis=0, keepdims=True)
                closest_ref[pl.ds(k, 1), :] = closest
                onehot = (sub512 == closest).astype(jnp.float32)
                xs = pts_ref[0, :, pl.ds(k * _CHUNK, _CHUNK)]
                return sums4 + jax.lax.dot_general(
                    onehot, xs, (((1,), (1,)), ((), ())),
                    preferred_element_type=jnp.float32)

            sums4 = jax.lax.fori_loop(
                0, _N_CHUNKS, chunk_body,
                jnp.zeros((_N_CLUSTERS, 4), jnp.float32))
            newc = sums4[:, 0:3] / (sums4[:, 3:4] + 1e-8)
            err_ref[0] = jnp.sum((newc - cent) ** 2)
            ct_ref[0] = newc

    jax.lax.fori_loop(0, _MAX_ITER, kmeans_iter, None)

    bv_ref[...] = jnp.full((_N_CLUSTERS, _BEST), _NEG, jnp.float32)
    bi_ref[...] = jnp.zeros((_N_CLUSTERS, _BEST), jnp.float32)
    cent = ct_ref[0]
    lane_best = jax.lax.broadcasted_iota(jnp.int32, (_N_CLUSTERS, _BEST), 1)
    lane_cand = jax.lax.broadcasted_iota(jnp.int32, (_N_CLUSTERS, _CAND), 1)

    def topk_chunk(k, _):
        s = scores_chunk(cent, k)
        closest = closest_ref[pl.ds(k, 1), :]
        ms = jnp.where(sub512 == closest, s, _NEG)
        bi0 = bi_ref[...]
        cand = jnp.concatenate([bv_ref[...], ms], axis=1)
        off = k * _CHUNK - _BEST

        def extract(t, carry):
            cand, nbv, nbi = carry
            m = jnp.max(cand, axis=1, keepdims=True)
            pos = jnp.min(jnp.where(cand == m, lane_cand, _CAND),
                          axis=1, keepdims=True)
            oldidx = jnp.sum(jnp.where(lane_best == pos, bi0, 0.0),
                             axis=1, keepdims=True)
            newidx = jnp.where(pos < _BEST, oldidx,
                               (pos + off).astype(jnp.float32))
            nbv = jnp.where(lane_best == t, m, nbv)
            nbi = jnp.where(lane_best == t, newidx, nbi)
            cand = jnp.where(lane_cand == pos, _DEAD, cand)
            return cand, nbv, nbi

        init = (cand,
                jnp.full((_N_CLUSTERS, _BEST), _NEG, jnp.float32),
                jnp.zeros((_N_CLUSTERS, _BEST), jnp.float32))
        _, nbv, nbi = jax.lax.fori_loop(0, _K_NEIGH, extract, init)
        bv_ref[...] = nbv
        bi_ref[...] = nbi

    jax.lax.fori_loop(0, _N_CHUNKS, topk_chunk, None)

    bv = bv_ref[...]
    bi = bi_ref[...]
    valid = bv > -1e29
    rowmax = jnp.max(bv, axis=1, keepdims=True)
    e = jnp.where(valid, jnp.exp(bv - rowmax), 0.0)
    w = e / jnp.maximum(jnp.sum(e, axis=1, keepdims=True), 1e-12)

    lane_f = jax.lax.broadcasted_iota(
        jnp.int32, (1, _CHUNK), 1).astype(jnp.float32)

    def feat_chunk(k, acc):
        jg = lane_f + (k * _CHUNK).astype(jnp.float32)
        wc = jnp.zeros((_N_CLUSTERS, _CHUNK), jnp.float32)
        for t in range(_K_NEIGH):
            wc = wc + jnp.where(bi[:, t:t + 1] == jg, w[:, t:t + 1], 0.0)
        f = feat_ref[0, :, pl.ds(k * _CHUNK, _CHUNK)]
        return acc + jax.lax.dot_general(
            f, wc, (((1,), (1,)), ((), ())),
            precision=jax.lax.Precision.HIGHEST,
            preferred_element_type=jnp.float32)

    out_ref[0] = jax.lax.fori_loop(
        0, _N_CHUNKS, feat_chunk,
        jnp.zeros((_F_DIM, _N_CLUSTERS), jnp.float32))


def _call(pts_t, features, centroids):
    b = pts_t.shape[0]
    return pl.pallas_call(
        _body,
        grid=(b,),
        in_specs=[
            pl.BlockSpec((1, 4, _N_POINTS), lambda i: (i, 0, 0)),
            pl.BlockSpec((1, _F_DIM, _N_POINTS), lambda i: (i, 0, 0)),
            pl.BlockSpec((1, _N_CLUSTERS, 3), lambda i: (i, 0, 0)),
        ],
        out_specs=(
            pl.BlockSpec((1, _N_CLUSTERS, 3), lambda i: (i, 0, 0)),
            pl.BlockSpec((1, _F_DIM, _N_CLUSTERS), lambda i: (i, 0, 0)),
        ),
        out_shape=(
            jax.ShapeDtypeStruct((b, _N_CLUSTERS, 3), jnp.float32),
            jax.ShapeDtypeStruct((b, _F_DIM, _N_CLUSTERS), jnp.float32),
        ),
        scratch_shapes=[
            pltpu.VMEM((_N_CHUNKS, _CHUNK), jnp.int32),
            pltpu.VMEM((_N_CLUSTERS, _BEST), jnp.float32),
            pltpu.VMEM((_N_CLUSTERS, _BEST), jnp.float32),
            pltpu.SMEM((1,), jnp.float32),
        ],
        compiler_params=pltpu.CompilerParams(
            dimension_semantics=("arbitrary",),
        ),
    )(pts_t, features, centroids)


def kernel(points, features, centroids):
    pts_t = jnp.transpose(points, (0, 2, 1))
    pts_t = jnp.concatenate(
        [pts_t, jnp.ones((pts_t.shape[0], 1, pts_t.shape[2]), jnp.float32)],
        axis=1)
    ct, agg = _call(pts_t, features, centroids)
    return ct, agg


# speedup vs baseline: 1.0746x; 1.0746x over previous
import jax
import jax.numpy as jnp
from jax.experimental import pallas as pl
from jax.experimental.pallas import tpu as pltpu

_N_CLUSTERS = 512
_MAX_ITER = 20
_TOL = 1e-4
_K_NEIGH = 15
_N_POINTS = 16384
_CHUNK = 4096
_N_CHUNKS = _N_POINTS // _CHUNK
_F_DIM = 128
_BEST = 128
_CAND = _BEST + _CHUNK
_NEG = -1e30
_DEAD = -2e30


def _body(pts_ref, feat_ref, cent0_ref, ct_ref, out_ref,
          closest_ref, bv_ref, bi_ref, err_ref):
    ct_ref[0] = cent0_ref[0]
    err_ref[0] = jnp.float32(1e30)

    sub512 = jax.lax.broadcasted_iota(jnp.int32, (_N_CLUSTERS, _CHUNK), 0)

    def scores_chunk(cent, k):
        off = k * _CHUNK
        xt = pts_ref[0, pl.ds(0, 3), pl.ds(off, _CHUNK)]
        dot = jax.lax.dot_general(cent, xt, (((1,), (0,)), ((), ())),
                                  preferred_element_type=jnp.float32)
        xn = (xt[0:1, :] * xt[0:1, :] + xt[2:3, :] * xt[2:3, :]) \
            + xt[1:2, :] * xt[1:2, :]
        cn = (cent[:, 0:1] * cent[:, 0:1] + cent[:, 2:3] * cent[:, 2:3]) \
            + cent[:, 1:2] * cent[:, 1:2]
        return (2.0 * dot - xn) - cn

    def kmeans_iter(it, _):
        @pl.when(err_ref[0] > _TOL)
        def _run():
            cent = ct_ref[0]

            def chunk_body(k, sums4):
                s = scores_chunk(cent, k)
                m = jnp.max(s, axis=0, keepdims=True)
                closest = jnp.min(
                    jnp.where(s >= m, sub512, _N_CLUSTERS),
                    axis=0, keepdims=True)
                closest_ref[pl.ds(k, 1), :] = closest
                onehot = (sub512 == closest).astype(jnp.float32)
                xs = pts_ref[0, :, pl.ds(k * _CHUNK, _CHUNK)]
                return sums4 + jax.lax.dot_general(
                    onehot, xs, (((1,), (1,)), ((), ())),
                    preferred_element_type=jnp.float32)

            sums4 = jnp.zeros((_N_CLUSTERS, 4), jnp.float32)
            for k in range(_N_CHUNKS):
                sums4 = chunk_body(k, sums4)
            newc = sums4[:, 0:3] / (sums4[:, 3:4] + 1e-8)
            err_ref[0] = jnp.sum((newc - cent) ** 2)
            ct_ref[0] = newc

    jax.lax.fori_loop(0, _MAX_ITER, kmeans_iter, None)

    bv_ref[...] = jnp.full((_N_CLUSTERS, _BEST), _NEG, jnp.float32)
    bi_ref[...] = jnp.zeros((_N_CLUSTERS, _BEST), jnp.float32)
    cent = ct_ref[0]
    lane_best = jax.lax.broadcasted_iota(jnp.int32, (_N_CLUSTERS, _BEST), 1)
    lane_cand = jax.lax.broadcasted_iota(jnp.int32, (_N_CLUSTERS, _CAND), 1)

    def topk_chunk(k, _):
        s = scores_chunk(cent, k)
        closest = closest_ref[pl.ds(k, 1), :]
        ms = jnp.where(sub512 == closest, s, _NEG)
        bi0 = bi_ref[...]
        cand = jnp.concatenate([bv_ref[...], ms], axis=1)
        off = k * _CHUNK - _BEST

        def extract(t, carry):
            cand, nbv, nbi = carry
            m = jnp.max(cand, axis=1, keepdims=True)
            pos = jnp.min(jnp.where(cand == m, lane_cand, _CAND),
                          axis=1, keepdims=True)
            oldidx = jnp.sum(jnp.where(lane_best == pos, bi0, 0.0),
                             axis=1, keepdims=True)
            newidx = jnp.where(pos < _BEST, oldidx,
                               (pos + off).astype(jnp.float32))
            nbv = jnp.where(lane_best == t, m, nbv)
            nbi = jnp.where(lane_best == t, newidx, nbi)
            cand = jnp.where(lane_cand == pos, _DEAD, cand)
            return cand, nbv, nbi

        init = (cand,
                jnp.full((_N_CLUSTERS, _BEST), _NEG, jnp.float32),
                jnp.zeros((_N_CLUSTERS, _BEST), jnp.float32))
        _, nbv, nbi = jax.lax.fori_loop(0, _K_NEIGH, extract, init)
        bv_ref[...] = nbv
        bi_ref[...] = nbi

    jax.lax.fori_loop(0, _N_CHUNKS, topk_chunk, None)

    bv = bv_ref[...]
    bi = bi_ref[...]
    valid = bv > -1e29
    rowmax = jnp.max(bv, axis=1, keepdims=True)
    e = jnp.where(valid, jnp.exp(bv - rowmax), 0.0)
    w = e / jnp.maximum(jnp.sum(e, axis=1, keepdims=True), 1e-12)

    lane_f = jax.lax.broadcasted_iota(
        jnp.int32, (1, _CHUNK), 1).astype(jnp.float32)

    def feat_chunk(k, acc):
        jg = lane_f + (k * _CHUNK).astype(jnp.float32)
        wc = jnp.zeros((_N_CLUSTERS, _CHUNK), jnp.float32)
        for t in range(_K_NEIGH):
            wc = wc + jnp.where(bi[:, t:t + 1] == jg, w[:, t:t + 1], 0.0)
        f = feat_ref[0, :, pl.ds(k * _CHUNK, _CHUNK)]
        return acc + jax.lax.dot_general(
            f, wc, (((1,), (1,)), ((), ())),
            precision=jax.lax.Precision.HIGHEST,
            preferred_element_type=jnp.float32)

    out_ref[0] = jax.lax.fori_loop(
        0, _N_CHUNKS, feat_chunk,
        jnp.zeros((_F_DIM, _N_CLUSTERS), jnp.float32))


def _call(pts_t, features, centroids):
    b = pts_t.shape[0]
    return pl.pallas_call(
        _body,
        grid=(b,),
        in_specs=[
            pl.BlockSpec((1, 4, _N_POINTS), lambda i: (i, 0, 0)),
            pl.BlockSpec((1, _F_DIM, _N_POINTS), lambda i: (i, 0, 0)),
            pl.BlockSpec((1, _N_CLUSTERS, 3), lambda i: (i, 0, 0)),
        ],
        out_specs=(
            pl.BlockSpec((1, _N_CLUSTERS, 3), lambda i: (i, 0, 0)),
            pl.BlockSpec((1, _F_DIM, _N_CLUSTERS), lambda i: (i, 0, 0)),
        ),
        out_shape=(
            jax.ShapeDtypeStruct((b, _N_CLUSTERS, 3), jnp.float32),
            jax.ShapeDtypeStruct((b, _F_DIM, _N_CLUSTERS), jnp.float32),
        ),
        scratch_shapes=[
            pltpu.VMEM((_N_CHUNKS, _CHUNK), jnp.int32),
            pltpu.VMEM((_N_CLUSTERS, _BEST), jnp.float32),
            pltpu.VMEM((_N_CLUSTERS, _BEST), jnp.float32),
            pltpu.SMEM((1,), jnp.float32),
        ],
        compiler_params=pltpu.CompilerParams(
            dimension_semantics=("arbitrary",),
        ),
    )(pts_t, features, centroids)


def kernel(points, features, centroids):
    pts_t = jnp.transpose(points, (0, 2, 1))
    pts_t = jnp.concatenate(
        [pts_t, jnp.ones((pts_t.shape[0], 1, pts_t.shape[2]), jnp.float32)],
        axis=1)
    ct, agg = _call(pts_t, features, centroids)
    return ct, agg


# speedup vs baseline: 1.0774x; 1.0027x over previous
import jax
import jax.numpy as jnp
from jax.experimental import pallas as pl
from jax.experimental.pallas import tpu as pltpu

_N_CLUSTERS = 512
_MAX_ITER = 20
_TOL = 1e-4
_K_NEIGH = 15
_N_POINTS = 16384
_CHUNK = 4096
_N_CHUNKS = _N_POINTS // _CHUNK
_F_DIM = 128
_BEST = 128
_CAND = _BEST + _CHUNK
_NEG = -1e30
_DEAD = -2e30


def _body(pts_ref, feat_ref, cent0_ref, ct_ref, out_ref,
          closest_ref, bv_ref, bi_ref, err_ref):
    ct_ref[0] = cent0_ref[0]
    err_ref[0] = jnp.float32(1e30)

    sub512 = jax.lax.broadcasted_iota(jnp.int32, (_N_CLUSTERS, _CHUNK), 0)

    def scores_chunk(cent, k):
        off = k * _CHUNK
        xt = pts_ref[0, pl.ds(0, 3), pl.ds(off, _CHUNK)]
        dot = jax.lax.dot_general(cent, xt, (((1,), (0,)), ((), ())),
                                  preferred_element_type=jnp.float32)
        xn = (xt[0:1, :] * xt[0:1, :] + xt[2:3, :] * xt[2:3, :]) \
            + xt[1:2, :] * xt[1:2, :]
        cn = (cent[:, 0:1] * cent[:, 0:1] + cent[:, 2:3] * cent[:, 2:3]) \
            + cent[:, 1:2] * cent[:, 1:2]
        return (2.0 * dot - xn) - cn

    def kmeans_iter(it, _):
        @pl.when(err_ref[0] > _TOL)
        def _run():
            cent = ct_ref[0]

            def chunk_body(k, sums4):
                s = scores_chunk(cent, k)
                m = jnp.max(s, axis=0, keepdims=True)
                closest = jnp.min(
                    jnp.where(s >= m, sub512, _N_CLUSTERS),
                    axis=0, keepdims=True)
                closest_ref[pl.ds(k, 1), :] = closest
                onehot = (sub512 == closest).astype(jnp.float32)
                xs = pts_ref[0, :, pl.ds(k * _CHUNK, _CHUNK)]
                return sums4 + jax.lax.dot_general(
                    onehot, xs, (((1,), (1,)), ((), ())),
                    preferred_element_type=jnp.float32)

            sums4 = jnp.zeros((_N_CLUSTERS, 4), jnp.float32)
            for k in range(_N_CHUNKS):
                sums4 = chunk_body(k, sums4)
            newc = sums4[:, 0:3] / (sums4[:, 3:4] + 1e-8)
            err_ref[0] = jnp.sum((newc - cent) ** 2)
            ct_ref[0] = newc

    jax.lax.fori_loop(0, _MAX_ITER, kmeans_iter, None)

    bv_ref[...] = jnp.full((_N_CLUSTERS, _BEST), _NEG, jnp.float32)
    bi_ref[...] = jnp.zeros((_N_CLUSTERS, _BEST), jnp.float32)
    cent = ct_ref[0]
    lane_best = jax.lax.broadcasted_iota(jnp.int32, (_N_CLUSTERS, _BEST), 1)
    lane_cand = jax.lax.broadcasted_iota(jnp.int32, (_N_CLUSTERS, _CAND), 1)

    def topk_chunk(k, _):
        s = scores_chunk(cent, k)
        closest = closest_ref[pl.ds(k, 1), :]
        ms = jnp.where(sub512 == closest, s, _NEG)
        bi0 = bi_ref[...]
        cand = jnp.concatenate([bv_ref[...], ms], axis=1)
        off = k * _CHUNK - _BEST

        def extract(t, carry):
            cand, nbv, nbi = carry
            m = jnp.max(cand, axis=1, keepdims=True)
            pos = jnp.min(jnp.where(cand == m, lane_cand, _CAND),
                          axis=1, keepdims=True)
            oldidx = jnp.sum(jnp.where(lane_best == pos, bi0, 0.0),
                             axis=1, keepdims=True)
            newidx = jnp.where(pos < _BEST, oldidx,
                               (pos + off).astype(jnp.float32))
            nbv = jnp.where(lane_best == t, m, nbv)
            nbi = jnp.where(lane_best == t, newidx, nbi)
            cand = jnp.where(lane_cand == pos, _DEAD, cand)
            return cand, nbv, nbi

        init = (cand,
                jnp.full((_N_CLUSTERS, _BEST), _NEG, jnp.float32),
                jnp.zeros((_N_CLUSTERS, _BEST), jnp.float32))
        _, nbv, nbi = jax.lax.fori_loop(0, _K_NEIGH, extract, init)
        bv_ref[...] = nbv
        bi_ref[...] = nbi

    for k in range(_N_CHUNKS):
        topk_chunk(k, None)

    bv = bv_ref[...]
    bi = bi_ref[...]
    valid = bv > -1e29
    rowmax = jnp.max(bv, axis=1, keepdims=True)
    e = jnp.where(valid, jnp.exp(bv - rowmax), 0.0)
    w = e / jnp.maximum(jnp.sum(e, axis=1, keepdims=True), 1e-12)

    lane_f = jax.lax.broadcasted_iota(
        jnp.int32, (1, _CHUNK), 1).astype(jnp.float32)

    def feat_chunk(k, acc):
        jg = lane_f + jnp.float32(k * _CHUNK)
        wc = jnp.zeros((_N_CLUSTERS, _CHUNK), jnp.float32)
        for t in range(_K_NEIGH):
            wc = wc + jnp.where(bi[:, t:t + 1] == jg, w[:, t:t + 1], 0.0)
        f = feat_ref[0, :, pl.ds(k * _CHUNK, _CHUNK)]
        return acc + jax.lax.dot_general(
            f, wc, (((1,), (1,)), ((), ())),
            precision=jax.lax.Precision.HIGHEST,
            preferred_element_type=jnp.float32)

    acc = jnp.zeros((_F_DIM, _N_CLUSTERS), jnp.float32)
    for k in range(_N_CHUNKS):
        acc = feat_chunk(k, acc)
    out_ref[0] = acc


def _call(pts_t, features, centroids):
    b = pts_t.shape[0]
    return pl.pallas_call(
        _body,
        grid=(b,),
        in_specs=[
            pl.BlockSpec((1, 4, _N_POINTS), lambda i: (i, 0, 0)),
            pl.BlockSpec((1, _F_DIM, _N_POINTS), lambda i: (i, 0, 0)),
            pl.BlockSpec((1, _N_CLUSTERS, 3), lambda i: (i, 0, 0)),
        ],
        out_specs=(
            pl.BlockSpec((1, _N_CLUSTERS, 3), lambda i: (i, 0, 0)),
            pl.BlockSpec((1, _F_DIM, _N_CLUSTERS), lambda i: (i, 0, 0)),
        ),
        out_shape=(
            jax.ShapeDtypeStruct((b, _N_CLUSTERS, 3), jnp.float32),
            jax.ShapeDtypeStruct((b, _F_DIM, _N_CLUSTERS), jnp.float32),
        ),
        scratch_shapes=[
            pltpu.VMEM((_N_CHUNKS, _CHUNK), jnp.int32),
            pltpu.VMEM((_N_CLUSTERS, _BEST), jnp.float32),
            pltpu.VMEM((_N_CLUSTERS, _BEST), jnp.float32),
            pltpu.SMEM((1,), jnp.float32),
        ],
        compiler_params=pltpu.CompilerParams(
            dimension_semantics=("arbitrary",),
        ),
    )(pts_t, features, centroids)


def kernel(points, features, centroids):
    pts_t = jnp.transpose(points, (0, 2, 1))
    pts_t = jnp.concatenate(
        [pts_t, jnp.ones((pts_t.shape[0], 1, pts_t.shape[2]), jnp.float32)],
        axis=1)
    ct, agg = _call(pts_t, features, centroids)
    return ct, agg
